# baseline (speedup 1.0000x reference)
import functools
import math

import jax
import jax.numpy as jnp
import numpy as np
from jax import lax
from jax.experimental import pallas as pl
from jax.experimental.pallas import tpu as pltpu
from jax.experimental.pallas import tpu_sc as plsc

D_MODEL = 768
CD = 192
PD = 64
SQRT_CD = math.sqrt(CD)

_DIV_NP = np.exp(
    np.arange(0, PD, 2, dtype=np.float64) * (-math.log(10000.0) / PD))
_A_ANG = np.arange(128, dtype=np.float64)[:, None] * 64.0 * _DIV_NP[None, :]
_C_ANG = np.arange(64, dtype=np.float64)[:, None] * _DIV_NP[None, :]
_SA_NP = np.sin(_A_ANG).astype(np.float32)
_CA_NP = np.cos(_A_ANG).astype(np.float32)
_SC_NP = np.sin(_C_ANG).astype(np.float32)
_CC_NP = np.cos(_C_ANG).astype(np.float32)


def _sc_gather_tokens(ids3, table):
    nw, n_ch, ch = ids3.shape
    width = table.shape[1]
    per_w = n_ch * ch
    t_total = nw * per_w
    info = plsc.get_sparse_core_info()
    nc = info.num_cores

    mesh = plsc.VectorSubcoreMesh(core_axis_name="c", subcore_axis_name="s")

    @functools.partial(
        pl.kernel,
        mesh=mesh,
        out_type=jax.ShapeDtypeStruct((t_total, width), jnp.float32),
        scratch_types=[
            pltpu.VMEM((n_ch, ch), jnp.int32),
            pltpu.VMEM((ch, width), jnp.float32),
            pltpu.VMEM((ch, width), jnp.float32),
            pltpu.SemaphoreType.DMA,
            pltpu.SemaphoreType.DMA,
        ],
    )
    def gather_kernel(ids_hbm, table_hbm, out_hbm, idx_v, buf0, buf1, sem0, sem1):
        wid = lax.axis_index("s") * nc + lax.axis_index("c")
        base = wid * per_w
        pltpu.sync_copy(ids_hbm.at[wid], idx_v)
        bufs = (buf0, buf1)
        sems = (sem0, sem1)
        copies = [None, None]
        for c in range(n_ch):
            copies[c % 2] = pltpu.async_copy(
                table_hbm.at[idx_v.at[c]], bufs[c % 2], sems[c % 2]
            )
            if c > 0:
                copies[(c - 1) % 2].wait()
                pltpu.sync_copy(
                    bufs[(c - 1) % 2],
                    out_hbm.at[pl.ds(base + (c - 1) * ch, ch)],
                )
        copies[(n_ch - 1) % 2].wait()
        pltpu.sync_copy(
            bufs[(n_ch - 1) % 2],
            out_hbm.at[pl.ds(base + (n_ch - 1) * ch, ch)],
        )

    return gather_kernel(ids3, table)


def _pack_body(src_ref, dst_ref):
    r = src_ref.shape[0]
    u = lax.bitcast_convert_type(src_ref[...], jnp.uint32)
    r16 = (u + 0x7FFF + ((u >> 16) & 1)) >> 16
    lo = r16[:, 0:128]
    hi = jnp.concatenate(
        [r16[:, 128:CD], jnp.zeros((r, 2 * 128 - CD), jnp.uint32)], axis=1)
    dst_ref[...] = lax.bitcast_convert_type(lo | (hi << 16), jnp.float32)


def _tc_pack_table(table):
    v = table.shape[0]
    rblk = 10000
    return pl.pallas_call(
        _pack_body,
        grid=(v // rblk,),
        in_specs=[pl.BlockSpec((rblk, CD), lambda i: (i, 0))],
        out_specs=pl.BlockSpec((rblk, 128), lambda i: (i, 0)),
        out_shape=jax.ShapeDtypeStruct((v, 128), jnp.float32),
        compiler_params=pltpu.CompilerParams(
            dimension_semantics=("arbitrary",)),
    )(table)


def _tc_body(tok_ref, tr_ref, pos_ref, vel_ref, beat_ref, bar_ref,
             track_t_ref, beat_t_ref, bar_t_ref, vel_t_ref,
             wt_ref, wt_tok_ref, we_ref, wo_ref, b_ref,
             sa_ref, ca_ref, sc_ref, cc_ref,
             out_ref, proj_ref):
    r = out_ref.shape[0]

    @pl.when(pl.program_id(0) == 0)
    def _init_proj():
        proj_ref[...] = jnp.zeros_like(proj_ref)
        proj_ref[pl.ds(0, 16), :] = jnp.dot(
            track_t_ref[...], wt_ref[pl.ds(CD, CD), :],
            preferred_element_type=jnp.float32)
        proj_ref[pl.ds(16, 8), :] = jnp.dot(
            beat_t_ref[...], wt_ref[pl.ds(2 * CD + PD, PD), :],
            preferred_element_type=jnp.float32)
        proj_ref[pl.ds(24, 16), :] = jnp.dot(
            bar_t_ref[...], wt_ref[pl.ds(2 * CD + 2 * PD, PD), :],
            preferred_element_type=jnp.float32)
        proj_ref[pl.ds(40, 32), :] = jnp.dot(
            vel_t_ref[...], wt_ref[pl.ds(3 * CD, CD), :],
            preferred_element_type=jnp.float32)

    u = lax.bitcast_convert_type(tok_ref[...], jnp.uint32)
    tok_lo = lax.bitcast_convert_type(u << 16, jnp.float32)
    tok_hi = lax.bitcast_convert_type(u & jnp.uint32(0xFFFF0000), jnp.float32)
    acc = (jnp.dot(tok_lo, wt_tok_ref[pl.ds(0, 128), :],
                   preferred_element_type=jnp.float32)
           + jnp.dot(tok_hi, wt_tok_ref[pl.ds(128, 128), :],
                     preferred_element_type=jnp.float32)) * SQRT_CD

    p = pos_ref[...]
    a_idx = p >> 6
    c_idx = p & 63
    oh_a = (lax.broadcasted_iota(jnp.int32, (r, 128), 1) == a_idx
            ).astype(jnp.float32)
    oh_c = (lax.broadcasted_iota(jnp.int32, (r, 64), 1) == c_idx
            ).astype(jnp.float32)
    s_a = jnp.dot(oh_a, sa_ref[...], preferred_element_type=jnp.float32)
    c_a = jnp.dot(oh_a, ca_ref[...], preferred_element_type=jnp.float32)
    s_c = jnp.dot(oh_c, sc_ref[...], preferred_element_type=jnp.float32)
    c_c = jnp.dot(oh_c, cc_ref[...], preferred_element_type=jnp.float32)
    sinv = s_a * c_c + c_a * s_c
    cosv = c_a * c_c - s_a * s_c
    acc += jnp.dot(sinv, we_ref[...], preferred_element_type=jnp.float32)
    acc += jnp.dot(cosv, wo_ref[...], preferred_element_type=jnp.float32)

    kio = lax.broadcasted_iota(jnp.int32, (r, 128), 1)
    oh = ((kio == tr_ref[...])
          | (kio == beat_ref[...] + 16)
          | (kio == bar_ref[...] + 24)
          | (kio == vel_ref[...] + 40)).astype(jnp.float32)
    acc += jnp.dot(oh, proj_ref[...], preferred_element_type=jnp.float32)

    out_ref[...] = acc + b_ref[...]


def _tc_body_prev(prev_ref, *refs):
    _tc_body(*refs)


def _tc_combine(tok_emb, track_ids, positions, velocity_bins, beat_positions,
                bar_positions, track_table, beat_p, bar_table, vel_table,
                wt, wt_tok, we, wo, b2, sa, ca, sc, cc,
                out_prev=None, blk0=0, t_total=None, interpret=False):
    t_chunk = tok_emb.shape[0]
    if t_total is None:
        t_total = t_chunk
    r = 512
    n_blk = t_chunk // r

    def row_spec(w):
        return pl.BlockSpec((r, w), lambda i: (i, 0))

    def full_spec(h, w):
        return pl.BlockSpec((h, w), lambda i: (0, 0))

    prev_specs = []
    prev_args = []
    aliases = {}
    body = _tc_body
    if out_prev is not None:
        prev_specs = [pl.BlockSpec(memory_space=pl.ANY)]
        prev_args = [out_prev]
        aliases = {0: 0}
        body = _tc_body_prev

    return pl.pallas_call(
        body,
        grid=(n_blk,),
        in_specs=prev_specs + [
            row_spec(128),
            row_spec(1),
            row_spec(1),
            row_spec(1),
            row_spec(1),
            row_spec(1),
            full_spec(16, CD),
            full_spec(8, PD),
            full_spec(16, PD),
            full_spec(32, CD),
            full_spec(D_MODEL, D_MODEL),
            full_spec(256, D_MODEL),
            full_spec(32, D_MODEL),
            full_spec(32, D_MODEL),
            full_spec(1, D_MODEL),
            full_spec(128, 32),
            full_spec(128, 32),
            full_spec(64, 32),
            full_spec(64, 32),
        ],
        out_specs=pl.BlockSpec((r, D_MODEL), lambda i: (blk0 + i, 0)),
        out_shape=jax.ShapeDtypeStruct((t_total, D_MODEL), jnp.float32),
        scratch_shapes=[pltpu.VMEM((128, D_MODEL), jnp.float32)],
        input_output_aliases=aliases,
        compiler_params=pltpu.CompilerParams(
            dimension_semantics=("arbitrary",)),
        interpret=interpret,
    )(*prev_args, tok_emb, track_ids, positions, velocity_bins, beat_positions,
      bar_positions, track_table, beat_p, bar_table, vel_table,
      wt, wt_tok, we, wo, b2, sa, ca, sc, cc)


def kernel(input_ids, track_ids, positions, velocity_bins, beat_positions,
           bar_positions, tok_table, track_table, beat_table, bar_table,
           vel_table, W, b):
    bsz, seq = input_ids.shape
    t_total = bsz * seq

    table_p = _tc_pack_table(tok_table)

    wt = W.T
    wt_tok = jnp.concatenate(
        [wt[0:CD], jnp.zeros((256 - CD, D_MODEL), jnp.float32)])
    pe_rows = wt[2 * CD:2 * CD + PD]
    we = pe_rows[0::2]
    wo = pe_rows[1::2]
    beat_p = jnp.concatenate(
        [beat_table, jnp.zeros((8 - beat_table.shape[0], PD), jnp.float32)])
    col = lambda x: x.reshape(t_total, 1).astype(jnp.int32)

    n_chunks = 4
    t_chunk = t_total // n_chunks
    ids_flat = input_ids.reshape(t_total).astype(jnp.int32)
    tr, po, ve, bt, br = (col(track_ids), col(positions), col(velocity_bins),
                          col(beat_positions), col(bar_positions))
    out = None
    for k in range(n_chunks):
        lo, hi = k * t_chunk, (k + 1) * t_chunk
        ids3 = ids_flat[lo:hi].reshape(32, t_chunk // (32 * 128), 128)
        tok_k = _sc_gather_tokens(ids3, table_p)
        out = _tc_combine(
            tok_k, tr[lo:hi], po[lo:hi], ve[lo:hi], bt[lo:hi], br[lo:hi],
            track_table, beat_p, bar_table, vel_table,
            wt, wt_tok, we, wo, b.reshape(1, D_MODEL),
            jnp.asarray(_SA_NP), jnp.asarray(_CA_NP),
            jnp.asarray(_SC_NP), jnp.asarray(_CC_NP),
            out_prev=out, blk0=k * (t_chunk // 512), t_total=t_total)
    return out.reshape(bsz, seq, D_MODEL)


# speedup vs baseline: 1.2637x; 1.2637x over previous
import functools
import math

import jax
import jax.numpy as jnp
import numpy as np
from jax import lax
from jax.experimental import pallas as pl
from jax.experimental.pallas import tpu as pltpu
from jax.experimental.pallas import tpu_sc as plsc

D_MODEL = 768
CD = 192
PD = 64
SQRT_CD = math.sqrt(CD)

_DIV_NP = np.exp(
    np.arange(0, PD, 2, dtype=np.float64) * (-math.log(10000.0) / PD))
_A_ANG = np.arange(128, dtype=np.float64)[:, None] * 64.0 * _DIV_NP[None, :]
_C_ANG = np.arange(64, dtype=np.float64)[:, None] * _DIV_NP[None, :]
_SA_NP = np.sin(_A_ANG).astype(np.float32)
_CA_NP = np.cos(_A_ANG).astype(np.float32)
_SC_NP = np.sin(_C_ANG).astype(np.float32)
_CC_NP = np.cos(_C_ANG).astype(np.float32)


def _sc_gather_tokens(ids3, table):
    nw, n_ch, ch = ids3.shape
    width = table.shape[1]
    per_w = n_ch * ch
    t_total = nw * per_w
    info = plsc.get_sparse_core_info()
    nc = info.num_cores

    mesh = plsc.VectorSubcoreMesh(core_axis_name="c", subcore_axis_name="s")

    @functools.partial(
        pl.kernel,
        mesh=mesh,
        out_type=jax.ShapeDtypeStruct((t_total, width), jnp.float32),
        scratch_types=[
            pltpu.VMEM((n_ch, ch), jnp.int32),
            pltpu.VMEM((ch, width), jnp.float32),
            pltpu.VMEM((ch, width), jnp.float32),
            pltpu.SemaphoreType.DMA,
            pltpu.SemaphoreType.DMA,
        ],
    )
    def gather_kernel(ids_hbm, table_hbm, out_hbm, idx_v, buf0, buf1, sem0, sem1):
        wid = lax.axis_index("s") * nc + lax.axis_index("c")
        base = wid * per_w
        pltpu.sync_copy(ids_hbm.at[wid], idx_v)
        bufs = (buf0, buf1)
        sems = (sem0, sem1)
        copies = [None, None]
        for c in range(n_ch):
            copies[c % 2] = pltpu.async_copy(
                table_hbm.at[idx_v.at[c]], bufs[c % 2], sems[c % 2]
            )
            if c > 0:
                copies[(c - 1) % 2].wait()
                pltpu.sync_copy(
                    bufs[(c - 1) % 2],
                    out_hbm.at[pl.ds(base + (c - 1) * ch, ch)],
                )
        copies[(n_ch - 1) % 2].wait()
        pltpu.sync_copy(
            bufs[(n_ch - 1) % 2],
            out_hbm.at[pl.ds(base + (n_ch - 1) * ch, ch)],
        )

    return gather_kernel(ids3, table)


def _pack_body(src_ref, dst_ref):
    r = src_ref.shape[0]
    u = lax.bitcast_convert_type(src_ref[...], jnp.uint32)
    r16 = (u + 0x7FFF + ((u >> 16) & 1)) >> 16
    lo = r16[:, 0:128]
    hi = jnp.concatenate(
        [r16[:, 128:CD], jnp.zeros((r, 2 * 128 - CD), jnp.uint32)], axis=1)
    dst_ref[...] = lax.bitcast_convert_type(lo | (hi << 16), jnp.float32)


def _tc_pack_table(table):
    v = table.shape[0]
    rblk = 10000
    return pl.pallas_call(
        _pack_body,
        grid=(v // rblk,),
        in_specs=[pl.BlockSpec((rblk, CD), lambda i: (i, 0))],
        out_specs=pl.BlockSpec((rblk, 128), lambda i: (i, 0)),
        out_shape=jax.ShapeDtypeStruct((v, 128), jnp.float32),
        compiler_params=pltpu.CompilerParams(
            dimension_semantics=("arbitrary",)),
    )(table)


def _tc_body(tok_ref, tr_ref, pos_ref, vel_ref, beat_ref, bar_ref,
             track_t_ref, beat_t_ref, bar_t_ref, vel_t_ref,
             wt_ref, wt_tok_ref, we_ref, wo_ref, b_ref,
             sa_ref, ca_ref, sc_ref, cc_ref,
             out_ref, proj_ref):
    r = out_ref.shape[0]

    @pl.when(pl.program_id(0) == 0)
    def _init_proj():
        proj_ref[...] = jnp.zeros_like(proj_ref)
        proj_ref[pl.ds(0, 16), :] = jnp.dot(
            track_t_ref[...], wt_ref[pl.ds(CD, CD), :],
            preferred_element_type=jnp.float32)
        proj_ref[pl.ds(16, 8), :] = jnp.dot(
            beat_t_ref[...], wt_ref[pl.ds(2 * CD + PD, PD), :],
            preferred_element_type=jnp.float32)
        proj_ref[pl.ds(24, 16), :] = jnp.dot(
            bar_t_ref[...], wt_ref[pl.ds(2 * CD + 2 * PD, PD), :],
            preferred_element_type=jnp.float32)
        proj_ref[pl.ds(40, 32), :] = jnp.dot(
            vel_t_ref[...], wt_ref[pl.ds(3 * CD, CD), :],
            preferred_element_type=jnp.float32)

    u = lax.bitcast_convert_type(tok_ref[...], jnp.uint32)
    tok_lo = lax.bitcast_convert_type(u << 16, jnp.float32)
    tok_hi = lax.bitcast_convert_type(u & jnp.uint32(0xFFFF0000), jnp.float32)
    acc = (jnp.dot(tok_lo, wt_tok_ref[pl.ds(0, 128), :],
                   preferred_element_type=jnp.float32)
           + jnp.dot(tok_hi, wt_tok_ref[pl.ds(128, 128), :],
                     preferred_element_type=jnp.float32)) * SQRT_CD

    p = pos_ref[...]
    a_idx = p >> 6
    c_idx = p & 63
    oh_a = (lax.broadcasted_iota(jnp.int32, (r, 128), 1) == a_idx
            ).astype(jnp.float32)
    oh_c = (lax.broadcasted_iota(jnp.int32, (r, 64), 1) == c_idx
            ).astype(jnp.float32)
    s_a = jnp.dot(oh_a, sa_ref[...], preferred_element_type=jnp.float32)
    c_a = jnp.dot(oh_a, ca_ref[...], preferred_element_type=jnp.float32)
    s_c = jnp.dot(oh_c, sc_ref[...], preferred_element_type=jnp.float32)
    c_c = jnp.dot(oh_c, cc_ref[...], preferred_element_type=jnp.float32)
    sinv = s_a * c_c + c_a * s_c
    cosv = c_a * c_c - s_a * s_c
    acc += jnp.dot(sinv, we_ref[...], preferred_element_type=jnp.float32)
    acc += jnp.dot(cosv, wo_ref[...], preferred_element_type=jnp.float32)

    kio = lax.broadcasted_iota(jnp.int32, (r, 128), 1)
    oh = ((kio == tr_ref[...])
          | (kio == beat_ref[...] + 16)
          | (kio == bar_ref[...] + 24)
          | (kio == vel_ref[...] + 40)).astype(jnp.float32)
    acc += jnp.dot(oh, proj_ref[...], preferred_element_type=jnp.float32)

    out_ref[...] = acc + b_ref[...]


def _tc_combine(tok_emb, track_ids, positions, velocity_bins, beat_positions,
                bar_positions, track_table, beat_p, bar_table, vel_table,
                wt, wt_tok, we, wo, b2, sa, ca, sc, cc, interpret=False):
    t_total = tok_emb.shape[0]
    r = 1024
    n_blk = t_total // r

    def row_spec(w):
        return pl.BlockSpec((r, w), lambda i: (i, 0))

    def full_spec(h, w):
        return pl.BlockSpec((h, w), lambda i: (0, 0))

    return pl.pallas_call(
        _tc_body,
        grid=(n_blk,),
        in_specs=[
            row_spec(128),
            row_spec(1),
            row_spec(1),
            row_spec(1),
            row_spec(1),
            row_spec(1),
            full_spec(16, CD),
            full_spec(8, PD),
            full_spec(16, PD),
            full_spec(32, CD),
            full_spec(D_MODEL, D_MODEL),
            full_spec(256, D_MODEL),
            full_spec(32, D_MODEL),
            full_spec(32, D_MODEL),
            full_spec(1, D_MODEL),
            full_spec(128, 32),
            full_spec(128, 32),
            full_spec(64, 32),
            full_spec(64, 32),
        ],
        out_specs=pl.BlockSpec((r, D_MODEL), lambda i: (i, 0)),
        out_shape=jax.ShapeDtypeStruct((t_total, D_MODEL), jnp.float32),
        scratch_shapes=[pltpu.VMEM((128, D_MODEL), jnp.float32)],
        compiler_params=pltpu.CompilerParams(
            dimension_semantics=("arbitrary",)),
        interpret=interpret,
    )(tok_emb, track_ids, positions, velocity_bins, beat_positions,
      bar_positions, track_table, beat_p, bar_table, vel_table,
      wt, wt_tok, we, wo, b2, sa, ca, sc, cc)


def kernel(input_ids, track_ids, positions, velocity_bins, beat_positions,
           bar_positions, tok_table, track_table, beat_table, bar_table,
           vel_table, W, b):
    bsz, seq = input_ids.shape
    t_total = bsz * seq

    ids3 = input_ids.reshape(32, t_total // (32 * 128), 128).astype(jnp.int32)
    table_p = _tc_pack_table(tok_table)
    tok_emb = _sc_gather_tokens(ids3, table_p)

    wt = W.T
    wt_tok = jnp.concatenate(
        [wt[0:CD], jnp.zeros((256 - CD, D_MODEL), jnp.float32)])
    pe_rows = wt[2 * CD:2 * CD + PD]
    we = pe_rows[0::2]
    wo = pe_rows[1::2]
    beat_p = jnp.concatenate(
        [beat_table, jnp.zeros((8 - beat_table.shape[0], PD), jnp.float32)])
    col = lambda x: x.reshape(t_total, 1).astype(jnp.int32)

    out = _tc_combine(
        tok_emb, col(track_ids), col(positions), col(velocity_bins),
        col(beat_positions), col(bar_positions),
        track_table, beat_p, bar_table, vel_table,
        wt, wt_tok, we, wo, b.reshape(1, D_MODEL),
        jnp.asarray(_SA_NP), jnp.asarray(_CA_NP),
        jnp.asarray(_SC_NP), jnp.asarray(_CC_NP))
    return out.reshape(bsz, seq, D_MODEL)


# speedup vs baseline: 1.2829x; 1.0152x over previous
import functools
import math

import jax
import jax.numpy as jnp
import numpy as np
from jax import lax
from jax.experimental import pallas as pl
from jax.experimental.pallas import tpu as pltpu
from jax.experimental.pallas import tpu_sc as plsc

D_MODEL = 768
CD = 192
PD = 64
SQRT_CD = math.sqrt(CD)

_DIV_NP = np.exp(
    np.arange(0, PD, 2, dtype=np.float64) * (-math.log(10000.0) / PD))
_A_ANG = np.arange(128, dtype=np.float64)[:, None] * 64.0 * _DIV_NP[None, :]
_C_ANG = np.arange(64, dtype=np.float64)[:, None] * _DIV_NP[None, :]
_SA_NP = np.sin(_A_ANG).astype(np.float32)
_CA_NP = np.cos(_A_ANG).astype(np.float32)
_SC_NP = np.sin(_C_ANG).astype(np.float32)
_CC_NP = np.cos(_C_ANG).astype(np.float32)


def _sc_gather_tokens(ids3, table):
    nw, n_ch, ch = ids3.shape
    width = table.shape[1]
    per_w = n_ch * ch
    t_total = nw * per_w
    info = plsc.get_sparse_core_info()
    nc = info.num_cores

    mesh = plsc.VectorSubcoreMesh(core_axis_name="c", subcore_axis_name="s")

    @functools.partial(
        pl.kernel,
        mesh=mesh,
        out_type=jax.ShapeDtypeStruct((t_total, width), jnp.float32),
        scratch_types=[
            pltpu.VMEM((n_ch, ch), jnp.int32),
            pltpu.VMEM((ch, width), jnp.float32),
            pltpu.VMEM((ch, width), jnp.float32),
            pltpu.SemaphoreType.DMA,
            pltpu.SemaphoreType.DMA,
        ],
    )
    def gather_kernel(ids_hbm, table_hbm, out_hbm, idx_v, buf0, buf1, sem0, sem1):
        wid = lax.axis_index("s") * nc + lax.axis_index("c")
        base = wid * per_w
        pltpu.sync_copy(ids_hbm.at[wid], idx_v)
        bufs = (buf0, buf1)
        sems = (sem0, sem1)
        copies = [None, None]
        for c in range(n_ch):
            copies[c % 2] = pltpu.async_copy(
                table_hbm.at[idx_v.at[c]], bufs[c % 2], sems[c % 2]
            )
            if c > 0:
                copies[(c - 1) % 2].wait()
                pltpu.sync_copy(
                    bufs[(c - 1) % 2],
                    out_hbm.at[pl.ds(base + (c - 1) * ch, ch)],
                )
        copies[(n_ch - 1) % 2].wait()
        pltpu.sync_copy(
            bufs[(n_ch - 1) % 2],
            out_hbm.at[pl.ds(base + (n_ch - 1) * ch, ch)],
        )

    return gather_kernel(ids3, table)


def _pack_body(src_ref, dst_ref):
    r = src_ref.shape[0]
    u = lax.bitcast_convert_type(src_ref[...], jnp.uint32)
    r16 = (u + 0x7FFF + ((u >> 16) & 1)) >> 16
    lo = r16[:, 0:128]
    hi = jnp.concatenate(
        [r16[:, 128:CD], jnp.zeros((r, 2 * 128 - CD), jnp.uint32)], axis=1)
    dst_ref[...] = lax.bitcast_convert_type(lo | (hi << 16), jnp.float32)


def _tc_pack_table(table):
    v = table.shape[0]
    rblk = 10000
    return pl.pallas_call(
        _pack_body,
        grid=(v // rblk,),
        in_specs=[pl.BlockSpec((rblk, CD), lambda i: (i, 0))],
        out_specs=pl.BlockSpec((rblk, 128), lambda i: (i, 0)),
        out_shape=jax.ShapeDtypeStruct((v, 128), jnp.float32),
        compiler_params=pltpu.CompilerParams(
            dimension_semantics=("arbitrary",)),
    )(table)


def _tc_body(tok_ref, tr_ref, pos_ref, vel_ref, beat_ref, bar_ref,
             track_t_ref, beat_t_ref, bar_t_ref, vel_t_ref,
             wt_ref, wt_tok_ref, we_ref, wo_ref, b_ref,
             sa_ref, ca_ref, sc_ref, cc_ref,
             out_ref, proj_ref):
    r = out_ref.shape[0]

    @pl.when(pl.program_id(0) == 0)
    def _init_proj():
        proj_ref[...] = jnp.zeros_like(proj_ref)
        proj_ref[pl.ds(0, 16), :] = jnp.dot(
            track_t_ref[...], wt_ref[pl.ds(CD, CD), :],
            preferred_element_type=jnp.float32)
        proj_ref[pl.ds(16, 8), :] = jnp.dot(
            beat_t_ref[...], wt_ref[pl.ds(2 * CD + PD, PD), :],
            preferred_element_type=jnp.float32)
        proj_ref[pl.ds(24, 16), :] = jnp.dot(
            bar_t_ref[...], wt_ref[pl.ds(2 * CD + 2 * PD, PD), :],
            preferred_element_type=jnp.float32)
        proj_ref[pl.ds(40, 32), :] = jnp.dot(
            vel_t_ref[...], wt_ref[pl.ds(3 * CD, CD), :],
            preferred_element_type=jnp.float32)

    u = lax.bitcast_convert_type(tok_ref[...], jnp.uint32)
    tok_lo = lax.bitcast_convert_type(u << 16, jnp.float32)
    tok_hi = lax.bitcast_convert_type(u & jnp.uint32(0xFFFF0000), jnp.float32)
    acc = (jnp.dot(tok_lo, wt_tok_ref[pl.ds(0, 128), :],
                   preferred_element_type=jnp.float32)
           + jnp.dot(tok_hi, wt_tok_ref[pl.ds(128, 128), :],
                     preferred_element_type=jnp.float32)) * SQRT_CD

    p = pos_ref[...]
    a_idx = p >> 6
    c_idx = p & 63
    oh_a = (lax.broadcasted_iota(jnp.int32, (r, 128), 1) == a_idx
            ).astype(jnp.float32)
    oh_c = (lax.broadcasted_iota(jnp.int32, (r, 64), 1) == c_idx
            ).astype(jnp.float32)
    s_a = jnp.dot(oh_a, sa_ref[...], preferred_element_type=jnp.float32)
    c_a = jnp.dot(oh_a, ca_ref[...], preferred_element_type=jnp.float32)
    s_c = jnp.dot(oh_c, sc_ref[...], preferred_element_type=jnp.float32)
    c_c = jnp.dot(oh_c, cc_ref[...], preferred_element_type=jnp.float32)
    sinv = s_a * c_c + c_a * s_c
    cosv = c_a * c_c - s_a * s_c
    acc += jnp.dot(sinv, we_ref[...], preferred_element_type=jnp.float32)
    acc += jnp.dot(cosv, wo_ref[...], preferred_element_type=jnp.float32)

    kio = lax.broadcasted_iota(jnp.int32, (r, 128), 1)
    oh = ((kio == tr_ref[...])
          | (kio == beat_ref[...] + 16)
          | (kio == bar_ref[...] + 24)
          | (kio == vel_ref[...] + 40)).astype(jnp.float32)
    acc += jnp.dot(oh, proj_ref[...], preferred_element_type=jnp.float32)

    out_ref[...] = acc + b_ref[...]


def _tc_combine(tok_emb, track_ids, positions, velocity_bins, beat_positions,
                bar_positions, track_table, beat_p, bar_table, vel_table,
                wt, wt_tok, we, wo, b2, sa, ca, sc, cc, interpret=False):
    t_total = tok_emb.shape[0]
    r = 2048
    n_blk = t_total // r

    def row_spec(w):
        return pl.BlockSpec((r, w), lambda i: (i, 0))

    def full_spec(h, w):
        return pl.BlockSpec((h, w), lambda i: (0, 0))

    return pl.pallas_call(
        _tc_body,
        grid=(n_blk,),
        in_specs=[
            row_spec(128),
            row_spec(1),
            row_spec(1),
            row_spec(1),
            row_spec(1),
            row_spec(1),
            full_spec(16, CD),
            full_spec(8, PD),
            full_spec(16, PD),
            full_spec(32, CD),
            full_spec(D_MODEL, D_MODEL),
            full_spec(256, D_MODEL),
            full_spec(32, D_MODEL),
            full_spec(32, D_MODEL),
            full_spec(1, D_MODEL),
            full_spec(128, 32),
            full_spec(128, 32),
            full_spec(64, 32),
            full_spec(64, 32),
        ],
        out_specs=pl.BlockSpec((r, D_MODEL), lambda i: (i, 0)),
        out_shape=jax.ShapeDtypeStruct((t_total, D_MODEL), jnp.float32),
        scratch_shapes=[pltpu.VMEM((128, D_MODEL), jnp.float32)],
        compiler_params=pltpu.CompilerParams(
            dimension_semantics=("arbitrary",)),
        interpret=interpret,
    )(tok_emb, track_ids, positions, velocity_bins, beat_positions,
      bar_positions, track_table, beat_p, bar_table, vel_table,
      wt, wt_tok, we, wo, b2, sa, ca, sc, cc)


def kernel(input_ids, track_ids, positions, velocity_bins, beat_positions,
           bar_positions, tok_table, track_table, beat_table, bar_table,
           vel_table, W, b):
    bsz, seq = input_ids.shape
    t_total = bsz * seq

    ids3 = input_ids.reshape(32, t_total // (32 * 128), 128).astype(jnp.int32)
    table_p = _tc_pack_table(tok_table)
    tok_emb = _sc_gather_tokens(ids3, table_p)

    wt = W.T
    wt_tok = jnp.concatenate(
        [wt[0:CD], jnp.zeros((256 - CD, D_MODEL), jnp.float32)])
    pe_rows = wt[2 * CD:2 * CD + PD]
    we = pe_rows[0::2]
    wo = pe_rows[1::2]
    beat_p = jnp.concatenate(
        [beat_table, jnp.zeros((8 - beat_table.shape[0], PD), jnp.float32)])
    col = lambda x: x.reshape(t_total, 1).astype(jnp.int32)

    out = _tc_combine(
        tok_emb, col(track_ids), col(positions), col(velocity_bins),
        col(beat_positions), col(bar_positions),
        track_table, beat_p, bar_table, vel_table,
        wt, wt_tok, we, wo, b.reshape(1, D_MODEL),
        jnp.asarray(_SA_NP), jnp.asarray(_CA_NP),
        jnp.asarray(_SC_NP), jnp.asarray(_CC_NP))
    return out.reshape(bsz, seq, D_MODEL)
